# baseline (speedup 1.0000x reference)
import jax
import jax.numpy as jnp
from jax import lax
from jax.experimental import pallas as pl
from jax.experimental.pallas import tpu as pltpu
from jax.experimental.pallas import tpu_sc as plsc

N_ROWS = 16384
N_COLS = 1000
BLOCK_ROWS = 128
NUM_BLOCKS = N_ROWS // BLOCK_ROWS
NUM_BINS = 5
LANES = 16
INV_TOTAL = 1.0 / float(N_ROWS * N_COLS)
BIG_IDX = 2 ** 30
PART_ROWS = NUM_BINS * 3


def _tc_rowstats_body(x_ref, lab_ref, conf_ref, acc_ref, bits_ref, cnt_ref):
    x = x_ref[...]
    rowmax = jnp.max(x, axis=1)
    conf_ref[...] = rowmax
    col = lax.broadcasted_iota(jnp.int32, x.shape, 1)
    ismax = x == rowmax[:, None]
    pred = jnp.min(jnp.where(ismax, col, BIG_IDX), axis=1)
    acc_ref[...] = (pred == lab_ref[...]).astype(jnp.float32)
    s = [jnp.sum((x > jnp.float32(k)).astype(jnp.float32), axis=1)
         for k in range(NUM_BINS + 1)]
    bits = jnp.zeros((BLOCK_ROWS,), jnp.int32)
    lane = lax.broadcasted_iota(jnp.int32, (1, 1, LANES), 2)
    cnt_row = jnp.zeros((1, 1, LANES), jnp.float32)
    for k in range(NUM_BINS):
        ck = s[k] - s[k + 1]
        bits = bits | ((ck > 0.0).astype(jnp.int32) << k)
        cnt_row = cnt_row + jnp.where(lane == k, jnp.sum(ck), 0.0)
    bits_ref[...] = bits
    cnt_ref[...] = cnt_row


_stage1 = pl.pallas_call(
    _tc_rowstats_body,
    grid=(NUM_BLOCKS,),
    in_specs=[
        pl.BlockSpec((BLOCK_ROWS, N_COLS), lambda i: (i, 0)),
        pl.BlockSpec((BLOCK_ROWS,), lambda i: (i,)),
    ],
    out_specs=[
        pl.BlockSpec((BLOCK_ROWS,), lambda i: (i,)),
        pl.BlockSpec((BLOCK_ROWS,), lambda i: (i,)),
        pl.BlockSpec((BLOCK_ROWS,), lambda i: (i,)),
        pl.BlockSpec((1, 1, LANES), lambda i: (i, 0, 0)),
    ],
    out_shape=[
        jax.ShapeDtypeStruct((N_ROWS,), jnp.float32),
        jax.ShapeDtypeStruct((N_ROWS,), jnp.float32),
        jax.ShapeDtypeStruct((N_ROWS,), jnp.int32),
        jax.ShapeDtypeStruct((NUM_BLOCKS, 1, LANES), jnp.float32),
    ],
)


def _make_sc_body(num_subcores, rows_per_tile):
    n_chunks = rows_per_tile // LANES

    def body(conf_hbm, acc_hbm, bits_hbm, cnts_hbm, out_hbm,
             conf_v, acc_v, bits_v, cnts_v, part_v, gath_v, out_v,
             shared):
        c = lax.axis_index("c")
        s = lax.axis_index("s")
        base = s * rows_per_tile
        pltpu.sync_copy(conf_hbm.at[pl.ds(base, rows_per_tile)], conf_v)
        pltpu.sync_copy(acc_hbm.at[pl.ds(base, rows_per_tile)], acc_v)
        pltpu.sync_copy(bits_hbm.at[pl.ds(base, rows_per_tile)], bits_v)

        zero = jnp.zeros((LANES,), jnp.float32)

        def row_step(j, carry):
            off = pl.multiple_of(j * LANES, LANES)
            cf = conf_v[pl.ds(off, LANES)]
            ac = acc_v[pl.ds(off, LANES)]
            bt = bits_v[pl.ds(off, LANES)]
            out = []
            for k in range(NUM_BINS):
                cntk, ak, sk = carry[k]
                mf = ((bt >> k) & 1).astype(jnp.float32)
                out.append((cntk + mf, ak + mf * ac, sk + mf * cf))
            return tuple(out)

        init = tuple((zero, zero, zero) for _ in range(NUM_BINS))
        stats = lax.fori_loop(0, n_chunks, row_step, init)
        for k in range(NUM_BINS):
            cntk, ak, sk = stats[k]
            part_v[pl.ds((3 * k + 0) * LANES, LANES)] = cntk
            part_v[pl.ds((3 * k + 1) * LANES, LANES)] = ak
            part_v[pl.ds((3 * k + 2) * LANES, LANES)] = sk
        part_v[pl.ds(PART_ROWS * LANES, LANES)] = zero
        pltpu.sync_copy(part_v, shared.at[s])
        plsc.subcore_barrier()

        pltpu.sync_copy(shared, gath_v)
        for r in range(PART_ROWS):
            v = zero
            for t in range(num_subcores):
                v = v + gath_v[t, pl.ds(r * LANES, LANES)]
            part_v[pl.ds(r * LANES, LANES)] = v

        pltpu.sync_copy(cnts_hbm, cnts_v)

        def cnt_step(j, acc):
            off = pl.multiple_of(j * LANES, LANES)
            return acc + cnts_v[pl.ds(off, LANES)]

        ecnt = lax.fori_loop(0, NUM_BLOCKS, cnt_step, zero)

        def lane_sum(r):
            v = part_v[pl.ds(r * LANES, LANES)]
            t = v[0]
            for l in range(1, LANES):
                t = t + v[l]
            return t

        lane = lax.iota(jnp.int32, LANES)
        cntv, asumv, csumv = zero, zero, zero
        for k in range(NUM_BINS):
            lk = lane == k
            cntv = jnp.where(lk, lane_sum(3 * k), cntv)
            asumv = jnp.where(lk, lane_sum(3 * k + 1), asumv)
            csumv = jnp.where(lk, lane_sum(3 * k + 2), csumv)

        safe = jnp.maximum(cntv, 1.0)
        accb = asumv / safe
        confb = csumv / safe
        term = jnp.abs(confb - accb) * (ecnt * jnp.float32(INV_TOTAL))
        term = jnp.where((ecnt > 0.0) & (lane < NUM_BINS), term, 0.0)
        ece = term[0]
        for k in range(1, NUM_BINS):
            ece = ece + term[k]
        out_v[...] = jnp.where(lane == 0, ece, 0.0)

        @pl.when(jnp.logical_and(c == 0, s == 0))
        def _():
            pltpu.sync_copy(out_v, out_hbm)

    return body


_stage2_cache = []


def _get_stage2():
    if not _stage2_cache:
        info = plsc.get_sparse_core_info()
        num_subcores = info.num_subcores
        rows_per_tile = N_ROWS // num_subcores
        _stage2_cache.append(pl.kernel(
            _make_sc_body(num_subcores, rows_per_tile),
            mesh=plsc.VectorSubcoreMesh(core_axis_name="c",
                                        subcore_axis_name="s"),
            out_type=jax.ShapeDtypeStruct((LANES,), jnp.float32),
            scratch_types=[
                pltpu.VMEM((rows_per_tile,), jnp.float32),
                pltpu.VMEM((rows_per_tile,), jnp.float32),
                pltpu.VMEM((rows_per_tile,), jnp.int32),
                pltpu.VMEM((NUM_BLOCKS * LANES,), jnp.float32),
                pltpu.VMEM((LANES * LANES,), jnp.float32),
                pltpu.VMEM((num_subcores, LANES * LANES), jnp.float32),
                pltpu.VMEM((LANES,), jnp.float32),
                pltpu.VMEM_SHARED((num_subcores, LANES * LANES), jnp.float32),
            ],
        ))
    return _stage2_cache[0]


def kernel(logits, labels):
    labels = labels.astype(jnp.int32)
    conf, acc, bits, cnts = _stage1(logits, labels)
    out = _get_stage2()(conf, acc, bits, cnts.reshape(NUM_BLOCKS * LANES))
    return out[:1]


# speedup vs baseline: 1.1143x; 1.1143x over previous
import jax
import jax.numpy as jnp
from jax import lax
from jax.experimental import pallas as pl
from jax.experimental.pallas import tpu as pltpu
from jax.experimental.pallas import tpu_sc as plsc

N_ROWS = 16384
N_COLS = 1000
BLOCK_ROWS = 256
NUM_BLOCKS = N_ROWS // BLOCK_ROWS
NUM_BINS = 5
LANES = 16
INV_TOTAL = 1.0 / float(N_ROWS * N_COLS)
BIG_IDX = 2 ** 30
PART_ROWS = NUM_BINS * 3


def _tc_rowstats_body(x_ref, lab_ref, conf_ref, acc_ref, bits_ref, cnt_ref):
    x = x_ref[...]
    rowmax = jnp.max(x, axis=1)
    conf_ref[...] = rowmax
    col = lax.broadcasted_iota(jnp.int32, x.shape, 1)
    ismax = x == rowmax[:, None]
    pred = jnp.min(jnp.where(ismax, col, BIG_IDX), axis=1)
    acc_ref[...] = (pred == lab_ref[...]).astype(jnp.float32)
    ones_col = jnp.ones((N_COLS,), jnp.bfloat16)
    s = []
    for k in range(NUM_BINS + 1):
        ind = (x > jnp.float32(k)).astype(jnp.bfloat16)
        s.append(lax.dot_general(
            ind, ones_col, (((1,), (0,)), ((), ())),
            preferred_element_type=jnp.float32))
    bits = jnp.zeros((BLOCK_ROWS,), jnp.int32)
    lane = lax.broadcasted_iota(jnp.int32, (1, 1, LANES), 2)
    cnt_row = jnp.zeros((1, 1, LANES), jnp.float32)
    for k in range(NUM_BINS):
        ck = s[k] - s[k + 1]
        bits = bits | ((ck > 0.0).astype(jnp.int32) << k)
        cnt_row = cnt_row + jnp.where(lane == k, jnp.sum(ck), 0.0)
    bits_ref[...] = bits
    cnt_ref[...] = cnt_row


_stage1 = pl.pallas_call(
    _tc_rowstats_body,
    grid=(NUM_BLOCKS,),
    in_specs=[
        pl.BlockSpec((BLOCK_ROWS, N_COLS), lambda i: (i, 0)),
        pl.BlockSpec((BLOCK_ROWS,), lambda i: (i,)),
    ],
    out_specs=[
        pl.BlockSpec((BLOCK_ROWS,), lambda i: (i,)),
        pl.BlockSpec((BLOCK_ROWS,), lambda i: (i,)),
        pl.BlockSpec((BLOCK_ROWS,), lambda i: (i,)),
        pl.BlockSpec((1, 1, LANES), lambda i: (i, 0, 0)),
    ],
    out_shape=[
        jax.ShapeDtypeStruct((N_ROWS,), jnp.float32),
        jax.ShapeDtypeStruct((N_ROWS,), jnp.float32),
        jax.ShapeDtypeStruct((N_ROWS,), jnp.int32),
        jax.ShapeDtypeStruct((NUM_BLOCKS, 1, LANES), jnp.float32),
    ],
)


def _make_sc_body(num_subcores, rows_per_tile):
    n_chunks = rows_per_tile // LANES

    def body(conf_hbm, acc_hbm, bits_hbm, cnts_hbm, out_hbm,
             conf_v, acc_v, bits_v, cnts_v, part_v, gath_v, out_v,
             shared):
        c = lax.axis_index("c")
        s = lax.axis_index("s")
        base = s * rows_per_tile
        pltpu.sync_copy(conf_hbm.at[pl.ds(base, rows_per_tile)], conf_v)
        pltpu.sync_copy(acc_hbm.at[pl.ds(base, rows_per_tile)], acc_v)
        pltpu.sync_copy(bits_hbm.at[pl.ds(base, rows_per_tile)], bits_v)

        zero = jnp.zeros((LANES,), jnp.float32)

        def row_step(j, carry):
            off = pl.multiple_of(j * LANES, LANES)
            cf = conf_v[pl.ds(off, LANES)]
            ac = acc_v[pl.ds(off, LANES)]
            bt = bits_v[pl.ds(off, LANES)]
            out = []
            for k in range(NUM_BINS):
                cntk, ak, sk = carry[k]
                mf = ((bt >> k) & 1).astype(jnp.float32)
                out.append((cntk + mf, ak + mf * ac, sk + mf * cf))
            return tuple(out)

        init = tuple((zero, zero, zero) for _ in range(NUM_BINS))
        stats = lax.fori_loop(0, n_chunks, row_step, init)
        for k in range(NUM_BINS):
            cntk, ak, sk = stats[k]
            part_v[pl.ds((3 * k + 0) * LANES, LANES)] = cntk
            part_v[pl.ds((3 * k + 1) * LANES, LANES)] = ak
            part_v[pl.ds((3 * k + 2) * LANES, LANES)] = sk
        part_v[pl.ds(PART_ROWS * LANES, LANES)] = zero
        pltpu.sync_copy(part_v, shared.at[s])
        plsc.subcore_barrier()

        pltpu.sync_copy(shared, gath_v)
        for r in range(PART_ROWS):
            v = zero
            for t in range(num_subcores):
                v = v + gath_v[t, pl.ds(r * LANES, LANES)]
            part_v[pl.ds(r * LANES, LANES)] = v

        pltpu.sync_copy(cnts_hbm, cnts_v)

        def cnt_step(j, acc):
            off = pl.multiple_of(j * LANES, LANES)
            return acc + cnts_v[pl.ds(off, LANES)]

        ecnt = lax.fori_loop(0, NUM_BLOCKS, cnt_step, zero)

        def lane_sum(r):
            v = part_v[pl.ds(r * LANES, LANES)]
            t = v[0]
            for l in range(1, LANES):
                t = t + v[l]
            return t

        lane = lax.iota(jnp.int32, LANES)
        cntv, asumv, csumv = zero, zero, zero
        for k in range(NUM_BINS):
            lk = lane == k
            cntv = jnp.where(lk, lane_sum(3 * k), cntv)
            asumv = jnp.where(lk, lane_sum(3 * k + 1), asumv)
            csumv = jnp.where(lk, lane_sum(3 * k + 2), csumv)

        safe = jnp.maximum(cntv, 1.0)
        accb = asumv / safe
        confb = csumv / safe
        term = jnp.abs(confb - accb) * (ecnt * jnp.float32(INV_TOTAL))
        term = jnp.where((ecnt > 0.0) & (lane < NUM_BINS), term, 0.0)
        ece = term[0]
        for k in range(1, NUM_BINS):
            ece = ece + term[k]
        out_v[...] = jnp.where(lane == 0, ece, 0.0)

        @pl.when(jnp.logical_and(c == 0, s == 0))
        def _():
            pltpu.sync_copy(out_v, out_hbm)

    return body


_stage2_cache = []


def _get_stage2():
    if not _stage2_cache:
        info = plsc.get_sparse_core_info()
        num_subcores = info.num_subcores
        rows_per_tile = N_ROWS // num_subcores
        _stage2_cache.append(pl.kernel(
            _make_sc_body(num_subcores, rows_per_tile),
            mesh=plsc.VectorSubcoreMesh(core_axis_name="c",
                                        subcore_axis_name="s"),
            out_type=jax.ShapeDtypeStruct((LANES,), jnp.float32),
            scratch_types=[
                pltpu.VMEM((rows_per_tile,), jnp.float32),
                pltpu.VMEM((rows_per_tile,), jnp.float32),
                pltpu.VMEM((rows_per_tile,), jnp.int32),
                pltpu.VMEM((NUM_BLOCKS * LANES,), jnp.float32),
                pltpu.VMEM((LANES * LANES,), jnp.float32),
                pltpu.VMEM((num_subcores, LANES * LANES), jnp.float32),
                pltpu.VMEM((LANES,), jnp.float32),
                pltpu.VMEM_SHARED((num_subcores, LANES * LANES), jnp.float32),
            ],
        ))
    return _stage2_cache[0]


def kernel(logits, labels):
    labels = labels.astype(jnp.int32)
    conf, acc, bits, cnts = _stage1(logits, labels)
    out = _get_stage2()(conf, acc, bits, cnts.reshape(NUM_BLOCKS * LANES))
    return out[:1]


# speedup vs baseline: 1.2022x; 1.0789x over previous
import jax
import jax.numpy as jnp
from jax import lax
from jax.experimental import pallas as pl
from jax.experimental.pallas import tpu as pltpu
from jax.experimental.pallas import tpu_sc as plsc

N_ROWS = 16384
N_COLS = 1000
BLOCK_ROWS = 512
NUM_BLOCKS = N_ROWS // BLOCK_ROWS
NUM_BINS = 5
LANES = 16
INV_TOTAL = 1.0 / float(N_ROWS * N_COLS)
BIG_IDX = 2 ** 30
PART_ROWS = NUM_BINS * 3


def _tc_rowstats_body(x_ref, lab_ref, conf_ref, acc_ref, bits_ref, cnt_ref):
    x = x_ref[...]
    rowmax = jnp.max(x, axis=1)
    conf_ref[...] = rowmax
    col = lax.broadcasted_iota(jnp.int32, x.shape, 1)
    ismax = x == rowmax[:, None]
    pred = jnp.min(jnp.where(ismax, col, BIG_IDX), axis=1)
    acc_ref[...] = (pred == lab_ref[...]).astype(jnp.float32)
    ones_col = jnp.ones((N_COLS,), jnp.bfloat16)
    s = []
    for k in range(NUM_BINS + 1):
        ind = (x > jnp.float32(k)).astype(jnp.bfloat16)
        s.append(lax.dot_general(
            ind, ones_col, (((1,), (0,)), ((), ())),
            preferred_element_type=jnp.float32))
    bits = jnp.zeros((BLOCK_ROWS,), jnp.int32)
    lane = lax.broadcasted_iota(jnp.int32, (1, 1, LANES), 2)
    cnt_row = jnp.zeros((1, 1, LANES), jnp.float32)
    for k in range(NUM_BINS):
        ck = s[k] - s[k + 1]
        bits = bits | ((ck > 0.0).astype(jnp.int32) << k)
        cnt_row = cnt_row + jnp.where(lane == k, jnp.sum(ck), 0.0)
    bits_ref[...] = bits
    cnt_ref[...] = cnt_row


_stage1 = pl.pallas_call(
    _tc_rowstats_body,
    grid=(NUM_BLOCKS,),
    in_specs=[
        pl.BlockSpec((BLOCK_ROWS, N_COLS), lambda i: (i, 0)),
        pl.BlockSpec((BLOCK_ROWS,), lambda i: (i,)),
    ],
    out_specs=[
        pl.BlockSpec((BLOCK_ROWS,), lambda i: (i,)),
        pl.BlockSpec((BLOCK_ROWS,), lambda i: (i,)),
        pl.BlockSpec((BLOCK_ROWS,), lambda i: (i,)),
        pl.BlockSpec((1, 1, LANES), lambda i: (i, 0, 0)),
    ],
    out_shape=[
        jax.ShapeDtypeStruct((N_ROWS,), jnp.float32),
        jax.ShapeDtypeStruct((N_ROWS,), jnp.float32),
        jax.ShapeDtypeStruct((N_ROWS,), jnp.int32),
        jax.ShapeDtypeStruct((NUM_BLOCKS, 1, LANES), jnp.float32),
    ],
)


def _make_sc_body(num_subcores, rows_per_tile):
    n_chunks = rows_per_tile // LANES

    def body(conf_hbm, acc_hbm, bits_hbm, cnts_hbm, out_hbm,
             conf_v, acc_v, bits_v, cnts_v, part_v, gath_v, out_v,
             shared):
        c = lax.axis_index("c")
        s = lax.axis_index("s")
        base = s * rows_per_tile
        pltpu.sync_copy(conf_hbm.at[pl.ds(base, rows_per_tile)], conf_v)
        pltpu.sync_copy(acc_hbm.at[pl.ds(base, rows_per_tile)], acc_v)
        pltpu.sync_copy(bits_hbm.at[pl.ds(base, rows_per_tile)], bits_v)

        zero = jnp.zeros((LANES,), jnp.float32)

        def row_step(j, carry):
            off = pl.multiple_of(j * LANES, LANES)
            cf = conf_v[pl.ds(off, LANES)]
            ac = acc_v[pl.ds(off, LANES)]
            bt = bits_v[pl.ds(off, LANES)]
            out = []
            for k in range(NUM_BINS):
                cntk, ak, sk = carry[k]
                mf = ((bt >> k) & 1).astype(jnp.float32)
                out.append((cntk + mf, ak + mf * ac, sk + mf * cf))
            return tuple(out)

        init = tuple((zero, zero, zero) for _ in range(NUM_BINS))
        stats = lax.fori_loop(0, n_chunks, row_step, init)
        for k in range(NUM_BINS):
            cntk, ak, sk = stats[k]
            part_v[pl.ds((3 * k + 0) * LANES, LANES)] = cntk
            part_v[pl.ds((3 * k + 1) * LANES, LANES)] = ak
            part_v[pl.ds((3 * k + 2) * LANES, LANES)] = sk
        part_v[pl.ds(PART_ROWS * LANES, LANES)] = zero
        pltpu.sync_copy(part_v, shared.at[s])
        plsc.subcore_barrier()

        pltpu.sync_copy(shared, gath_v)
        for r in range(PART_ROWS):
            v = zero
            for t in range(num_subcores):
                v = v + gath_v[t, pl.ds(r * LANES, LANES)]
            part_v[pl.ds(r * LANES, LANES)] = v

        pltpu.sync_copy(cnts_hbm, cnts_v)

        def cnt_step(j, acc):
            off = pl.multiple_of(j * LANES, LANES)
            return acc + cnts_v[pl.ds(off, LANES)]

        ecnt = lax.fori_loop(0, NUM_BLOCKS, cnt_step, zero)

        def lane_sum(r):
            v = part_v[pl.ds(r * LANES, LANES)]
            t = v[0]
            for l in range(1, LANES):
                t = t + v[l]
            return t

        lane = lax.iota(jnp.int32, LANES)
        cntv, asumv, csumv = zero, zero, zero
        for k in range(NUM_BINS):
            lk = lane == k
            cntv = jnp.where(lk, lane_sum(3 * k), cntv)
            asumv = jnp.where(lk, lane_sum(3 * k + 1), asumv)
            csumv = jnp.where(lk, lane_sum(3 * k + 2), csumv)

        safe = jnp.maximum(cntv, 1.0)
        accb = asumv / safe
        confb = csumv / safe
        term = jnp.abs(confb - accb) * (ecnt * jnp.float32(INV_TOTAL))
        term = jnp.where((ecnt > 0.0) & (lane < NUM_BINS), term, 0.0)
        ece = term[0]
        for k in range(1, NUM_BINS):
            ece = ece + term[k]
        out_v[...] = jnp.where(lane == 0, ece, 0.0)

        @pl.when(jnp.logical_and(c == 0, s == 0))
        def _():
            pltpu.sync_copy(out_v, out_hbm)

    return body


_stage2_cache = []


def _get_stage2():
    if not _stage2_cache:
        info = plsc.get_sparse_core_info()
        num_subcores = info.num_subcores
        rows_per_tile = N_ROWS // num_subcores
        _stage2_cache.append(pl.kernel(
            _make_sc_body(num_subcores, rows_per_tile),
            mesh=plsc.VectorSubcoreMesh(core_axis_name="c",
                                        subcore_axis_name="s",
                                        num_cores=1),
            out_type=jax.ShapeDtypeStruct((LANES,), jnp.float32),
            scratch_types=[
                pltpu.VMEM((rows_per_tile,), jnp.float32),
                pltpu.VMEM((rows_per_tile,), jnp.float32),
                pltpu.VMEM((rows_per_tile,), jnp.int32),
                pltpu.VMEM((NUM_BLOCKS * LANES,), jnp.float32),
                pltpu.VMEM((LANES * LANES,), jnp.float32),
                pltpu.VMEM((num_subcores, LANES * LANES), jnp.float32),
                pltpu.VMEM((LANES,), jnp.float32),
                pltpu.VMEM_SHARED((num_subcores, LANES * LANES), jnp.float32),
            ],
        ))
    return _stage2_cache[0]


def kernel(logits, labels):
    labels = labels.astype(jnp.int32)
    conf, acc, bits, cnts = _stage1(logits, labels)
    out = _get_stage2()(conf, acc, bits, cnts.reshape(NUM_BLOCKS * LANES))
    return out[:1]


# speedup vs baseline: 1.2062x; 1.0033x over previous
import jax
import jax.numpy as jnp
from jax import lax
from jax.experimental import pallas as pl
from jax.experimental.pallas import tpu as pltpu
from jax.experimental.pallas import tpu_sc as plsc

N_ROWS = 16384
N_COLS = 1000
BLOCK_ROWS = 512
NUM_BLOCKS = N_ROWS // BLOCK_ROWS
NUM_BINS = 5
LANES = 16
INV_TOTAL = 1.0 / float(N_ROWS * N_COLS)
BIG_IDX = 2 ** 30
PART_ROWS = NUM_BINS * 3


def _tc_rowstats_body(x_ref, lab_ref, conf_ref, acc_ref, bits_ref, cnt_ref):
    x = x_ref[...]
    rowmax = jnp.max(x, axis=1)
    conf_ref[...] = rowmax
    col = lax.broadcasted_iota(jnp.int32, x.shape, 1)
    ismax = x == rowmax[:, None]
    pred = jnp.min(jnp.where(ismax, col, BIG_IDX), axis=1)
    acc_ref[...] = (pred == lab_ref[...]).astype(jnp.float32)
    ones_col = jnp.ones((N_COLS,), jnp.bfloat16)
    s = []
    for k in range(NUM_BINS + 1):
        ind = (x > jnp.float32(k)).astype(jnp.bfloat16)
        s.append(lax.dot_general(
            ind, ones_col, (((1,), (0,)), ((), ())),
            preferred_element_type=jnp.float32))
    bits = jnp.zeros((BLOCK_ROWS,), jnp.int32)
    lane = lax.broadcasted_iota(jnp.int32, (1, 1, LANES), 2)
    cnt_row = jnp.zeros((1, 1, LANES), jnp.float32)
    for k in range(NUM_BINS):
        ck = s[k] - s[k + 1]
        bits = bits | ((ck > 0.0).astype(jnp.int32) << k)
        cnt_row = cnt_row + jnp.where(lane == k, jnp.sum(ck), 0.0)
    bits_ref[...] = bits
    cnt_ref[...] = cnt_row


_stage1 = pl.pallas_call(
    _tc_rowstats_body,
    grid=(NUM_BLOCKS,),
    in_specs=[
        pl.BlockSpec((BLOCK_ROWS, N_COLS), lambda i: (i, 0)),
        pl.BlockSpec((BLOCK_ROWS,), lambda i: (i,)),
    ],
    out_specs=[
        pl.BlockSpec((BLOCK_ROWS,), lambda i: (i,)),
        pl.BlockSpec((BLOCK_ROWS,), lambda i: (i,)),
        pl.BlockSpec((BLOCK_ROWS,), lambda i: (i,)),
        pl.BlockSpec((1, 1, LANES), lambda i: (i, 0, 0)),
    ],
    out_shape=[
        jax.ShapeDtypeStruct((N_ROWS,), jnp.float32),
        jax.ShapeDtypeStruct((N_ROWS,), jnp.float32),
        jax.ShapeDtypeStruct((N_ROWS,), jnp.int32),
        jax.ShapeDtypeStruct((NUM_BLOCKS, 1, LANES), jnp.float32),
    ],
)


def _make_sc_body(num_subcores, rows_per_tile):
    n_chunks = rows_per_tile // LANES

    def body(conf_hbm, acc_hbm, bits_hbm, cnts_hbm, out_hbm,
             conf_v, acc_v, bits_v, cnts_v, part_v, gath_v, out_v,
             shared):
        c = lax.axis_index("c")
        s = lax.axis_index("s")
        base = s * rows_per_tile
        pltpu.sync_copy(conf_hbm.at[pl.ds(base, rows_per_tile)], conf_v)
        pltpu.sync_copy(acc_hbm.at[pl.ds(base, rows_per_tile)], acc_v)
        pltpu.sync_copy(bits_hbm.at[pl.ds(base, rows_per_tile)], bits_v)

        zero = jnp.zeros((LANES,), jnp.float32)

        def row_step(j, carry):
            off = pl.multiple_of(j * LANES, LANES)
            cf = conf_v[pl.ds(off, LANES)]
            ac = acc_v[pl.ds(off, LANES)]
            bt = bits_v[pl.ds(off, LANES)]
            out = []
            for k in range(NUM_BINS):
                cntk, ak, sk = carry[k]
                mf = ((bt >> k) & 1).astype(jnp.float32)
                out.append((cntk + mf, ak + mf * ac, sk + mf * cf))
            return tuple(out)

        init = tuple((zero, zero, zero) for _ in range(NUM_BINS))
        stats = lax.fori_loop(0, n_chunks, row_step, init)
        for k in range(NUM_BINS):
            cntk, ak, sk = stats[k]
            part_v[pl.ds((3 * k + 0) * LANES, LANES)] = cntk
            part_v[pl.ds((3 * k + 1) * LANES, LANES)] = ak
            part_v[pl.ds((3 * k + 2) * LANES, LANES)] = sk
        part_v[pl.ds(PART_ROWS * LANES, LANES)] = zero
        pltpu.sync_copy(part_v, shared.at[s])
        plsc.subcore_barrier()

        pltpu.sync_copy(shared, gath_v)
        for r in range(PART_ROWS):
            v = zero
            for t in range(num_subcores):
                v = v + gath_v[t, pl.ds(r * LANES, LANES)]
            part_v[pl.ds(r * LANES, LANES)] = v

        pltpu.sync_copy(cnts_hbm, cnts_v)

        def cnt_step(j, acc):
            off = pl.multiple_of(j * LANES, LANES)
            return acc + cnts_v[pl.ds(off, LANES)]

        ecnt = lax.fori_loop(0, NUM_BLOCKS, cnt_step, zero)

        def lane_sum(r):
            v = part_v[pl.ds(r * LANES, LANES)]
            t = v[0]
            for l in range(1, LANES):
                t = t + v[l]
            return t

        lane = lax.iota(jnp.int32, LANES)
        cntv, asumv, csumv = zero, zero, zero
        for k in range(NUM_BINS):
            lk = lane == k
            cntv = jnp.where(lk, lane_sum(3 * k + 0), cntv)
            asumv = jnp.where(lk, lane_sum(3 * k + 1), asumv)
            csumv = jnp.where(lk, lane_sum(3 * k + 2), csumv)

        safe = jnp.maximum(cntv, 1.0)
        accb = asumv / safe
        confb = csumv / safe
        term = jnp.abs(confb - accb) * (ecnt * jnp.float32(INV_TOTAL))
        term = jnp.where((ecnt > 0.0) & (lane < NUM_BINS), term, 0.0)
        ece = term[0]
        for k in range(1, NUM_BINS):
            ece = ece + term[k]
        out_v[...] = jnp.where(lane == 0, ece, 0.0)

        @pl.when(jnp.logical_and(c == 0, s == 0))
        def _():
            pltpu.sync_copy(out_v, out_hbm)

    return body


_stage2_cache = []


def _get_stage2():
    if not _stage2_cache:
        info = plsc.get_sparse_core_info()
        num_subcores = info.num_subcores
        rows_per_tile = N_ROWS // num_subcores
        _stage2_cache.append(pl.kernel(
            _make_sc_body(num_subcores, rows_per_tile),
            mesh=plsc.VectorSubcoreMesh(core_axis_name="c",
                                        subcore_axis_name="s",
                                        num_cores=1),
            out_type=jax.ShapeDtypeStruct((LANES,), jnp.float32),
            scratch_types=[
                pltpu.VMEM((rows_per_tile,), jnp.float32),
                pltpu.VMEM((rows_per_tile,), jnp.float32),
                pltpu.VMEM((rows_per_tile,), jnp.int32),
                pltpu.VMEM((NUM_BLOCKS * LANES,), jnp.float32),
                pltpu.VMEM(((PART_ROWS + 1) * LANES,), jnp.float32),
                pltpu.VMEM((num_subcores, (PART_ROWS + 1) * LANES),
                           jnp.float32),
                pltpu.VMEM((LANES,), jnp.float32),
                pltpu.VMEM_SHARED((num_subcores, (PART_ROWS + 1) * LANES),
                                  jnp.float32),
            ],
        ))
    return _stage2_cache[0]


def kernel(logits, labels):
    labels = labels.astype(jnp.int32)
    conf, acc, bits, cnts = _stage1(logits, labels)
    out = _get_stage2()(conf, acc, bits, cnts.reshape(NUM_BLOCKS * LANES))
    return out[:1]


# speedup vs baseline: 1.2330x; 1.0222x over previous
import jax
import jax.numpy as jnp
from jax import lax
from jax.experimental import pallas as pl
from jax.experimental.pallas import tpu as pltpu
from jax.experimental.pallas import tpu_sc as plsc

N_ROWS = 16384
N_COLS = 1000
BLOCK_ROWS = 1024
NUM_BLOCKS = N_ROWS // BLOCK_ROWS
NUM_BINS = 5
LANES = 16
INV_TOTAL = 1.0 / float(N_ROWS * N_COLS)
BIG_IDX = 2 ** 30
PART_ROWS = NUM_BINS * 3


def _tc_rowstats_body(x_ref, lab_ref, conf_ref, acc_ref, bits_ref, cnt_ref):
    x = x_ref[...]
    rowmax = jnp.max(x, axis=1)
    conf_ref[...] = rowmax
    col = lax.broadcasted_iota(jnp.int32, x.shape, 1)
    ismax = x == rowmax[:, None]
    pred = jnp.min(jnp.where(ismax, col, BIG_IDX), axis=1)
    acc_ref[...] = (pred == lab_ref[...]).astype(jnp.float32)
    ones_col = jnp.ones((N_COLS,), jnp.bfloat16)
    s = []
    for k in range(NUM_BINS + 1):
        ind = (x > jnp.float32(k)).astype(jnp.bfloat16)
        s.append(lax.dot_general(
            ind, ones_col, (((1,), (0,)), ((), ())),
            preferred_element_type=jnp.float32))
    bits = jnp.zeros((BLOCK_ROWS,), jnp.int32)
    lane = lax.broadcasted_iota(jnp.int32, (1, 1, LANES), 2)
    cnt_row = jnp.zeros((1, 1, LANES), jnp.float32)
    for k in range(NUM_BINS):
        ck = s[k] - s[k + 1]
        bits = bits | ((ck > 0.0).astype(jnp.int32) << k)
        cnt_row = cnt_row + jnp.where(lane == k, jnp.sum(ck), 0.0)
    bits_ref[...] = bits
    cnt_ref[...] = cnt_row


_stage1 = pl.pallas_call(
    _tc_rowstats_body,
    grid=(NUM_BLOCKS,),
    in_specs=[
        pl.BlockSpec((BLOCK_ROWS, N_COLS), lambda i: (i, 0)),
        pl.BlockSpec((BLOCK_ROWS,), lambda i: (i,)),
    ],
    out_specs=[
        pl.BlockSpec((BLOCK_ROWS,), lambda i: (i,)),
        pl.BlockSpec((BLOCK_ROWS,), lambda i: (i,)),
        pl.BlockSpec((BLOCK_ROWS,), lambda i: (i,)),
        pl.BlockSpec((1, 1, LANES), lambda i: (i, 0, 0)),
    ],
    out_shape=[
        jax.ShapeDtypeStruct((N_ROWS,), jnp.float32),
        jax.ShapeDtypeStruct((N_ROWS,), jnp.float32),
        jax.ShapeDtypeStruct((N_ROWS,), jnp.int32),
        jax.ShapeDtypeStruct((NUM_BLOCKS, 1, LANES), jnp.float32),
    ],
)


def _make_sc_body(num_subcores, rows_per_tile):
    n_chunks = rows_per_tile // LANES

    def body(conf_hbm, acc_hbm, bits_hbm, cnts_hbm, out_hbm,
             conf_v, acc_v, bits_v, cnts_v, part_v, gath_v, out_v,
             shared):
        c = lax.axis_index("c")
        s = lax.axis_index("s")
        base = s * rows_per_tile
        pltpu.sync_copy(conf_hbm.at[pl.ds(base, rows_per_tile)], conf_v)
        pltpu.sync_copy(acc_hbm.at[pl.ds(base, rows_per_tile)], acc_v)
        pltpu.sync_copy(bits_hbm.at[pl.ds(base, rows_per_tile)], bits_v)

        zero = jnp.zeros((LANES,), jnp.float32)

        def row_step(j, carry):
            off = pl.multiple_of(j * LANES, LANES)
            cf = conf_v[pl.ds(off, LANES)]
            ac = acc_v[pl.ds(off, LANES)]
            bt = bits_v[pl.ds(off, LANES)]
            out = []
            for k in range(NUM_BINS):
                cntk, ak, sk = carry[k]
                mf = ((bt >> k) & 1).astype(jnp.float32)
                out.append((cntk + mf, ak + mf * ac, sk + mf * cf))
            return tuple(out)

        init = tuple((zero, zero, zero) for _ in range(NUM_BINS))
        stats = lax.fori_loop(0, n_chunks, row_step, init)
        for k in range(NUM_BINS):
            cntk, ak, sk = stats[k]
            part_v[pl.ds((3 * k + 0) * LANES, LANES)] = cntk
            part_v[pl.ds((3 * k + 1) * LANES, LANES)] = ak
            part_v[pl.ds((3 * k + 2) * LANES, LANES)] = sk
        part_v[pl.ds(PART_ROWS * LANES, LANES)] = zero
        pltpu.sync_copy(part_v, shared.at[s])
        plsc.subcore_barrier()

        pltpu.sync_copy(shared, gath_v)
        for r in range(PART_ROWS):
            v = zero
            for t in range(num_subcores):
                v = v + gath_v[t, pl.ds(r * LANES, LANES)]
            part_v[pl.ds(r * LANES, LANES)] = v

        pltpu.sync_copy(cnts_hbm, cnts_v)

        def cnt_step(j, acc):
            off = pl.multiple_of(j * LANES, LANES)
            return acc + cnts_v[pl.ds(off, LANES)]

        ecnt = lax.fori_loop(0, NUM_BLOCKS, cnt_step, zero)

        def lane_sum(r):
            v = part_v[pl.ds(r * LANES, LANES)]
            t = v[0]
            for l in range(1, LANES):
                t = t + v[l]
            return t

        lane = lax.iota(jnp.int32, LANES)
        cntv, asumv, csumv = zero, zero, zero
        for k in range(NUM_BINS):
            lk = lane == k
            cntv = jnp.where(lk, lane_sum(3 * k + 0), cntv)
            asumv = jnp.where(lk, lane_sum(3 * k + 1), asumv)
            csumv = jnp.where(lk, lane_sum(3 * k + 2), csumv)

        safe = jnp.maximum(cntv, 1.0)
        accb = asumv / safe
        confb = csumv / safe
        term = jnp.abs(confb - accb) * (ecnt * jnp.float32(INV_TOTAL))
        term = jnp.where((ecnt > 0.0) & (lane < NUM_BINS), term, 0.0)
        ece = term[0]
        for k in range(1, NUM_BINS):
            ece = ece + term[k]
        out_v[...] = jnp.where(lane == 0, ece, 0.0)

        @pl.when(jnp.logical_and(c == 0, s == 0))
        def _():
            pltpu.sync_copy(out_v, out_hbm)

    return body


_stage2_cache = []


def _get_stage2():
    if not _stage2_cache:
        info = plsc.get_sparse_core_info()
        num_subcores = info.num_subcores
        rows_per_tile = N_ROWS // num_subcores
        _stage2_cache.append(pl.kernel(
            _make_sc_body(num_subcores, rows_per_tile),
            mesh=plsc.VectorSubcoreMesh(core_axis_name="c",
                                        subcore_axis_name="s",
                                        num_cores=1),
            out_type=jax.ShapeDtypeStruct((LANES,), jnp.float32),
            scratch_types=[
                pltpu.VMEM((rows_per_tile,), jnp.float32),
                pltpu.VMEM((rows_per_tile,), jnp.float32),
                pltpu.VMEM((rows_per_tile,), jnp.int32),
                pltpu.VMEM((NUM_BLOCKS * LANES,), jnp.float32),
                pltpu.VMEM(((PART_ROWS + 1) * LANES,), jnp.float32),
                pltpu.VMEM((num_subcores, (PART_ROWS + 1) * LANES),
                           jnp.float32),
                pltpu.VMEM((LANES,), jnp.float32),
                pltpu.VMEM_SHARED((num_subcores, (PART_ROWS + 1) * LANES),
                                  jnp.float32),
            ],
        ))
    return _stage2_cache[0]


def kernel(logits, labels):
    labels = labels.astype(jnp.int32)
    conf, acc, bits, cnts = _stage1(logits, labels)
    out = _get_stage2()(conf, acc, bits, cnts.reshape(NUM_BLOCKS * LANES))
    return out[:1]
